# baseline (speedup 1.0000x reference)
import functools

import jax
import jax.numpy as jnp
from jax import lax
from jax.experimental import pallas as pl
from jax.experimental.pallas import tpu as pltpu
from jax.experimental.pallas import tpu_sc as plsc

_NC = 2
_NS = 16
_NW = _NC * _NS
_R = 16


def _sc_body(row, bpw, cls_hbm, emb_hbm, out_hbm, buf, sem):
    wid = lax.axis_index("s") * _NC + lax.axis_index("c")
    d = 64
    for r in range(_R):
        pltpu.sync_copy(cls_hbm, buf.at[pl.ds(r * row, d)])
        pltpu.sync_copy(emb_hbm, buf.at[pl.ds(r * row + d, row - d)])
    base = wid * bpw * row
    chunk = _R * row
    copies = [
        pltpu.make_async_copy(buf, out_hbm.at[pl.ds(base + c * chunk, chunk)], sem)
        for c in range(bpw // _R)
    ]
    for cp in copies:
        cp.start()
    for cp in copies:
        cp.wait()


def kernel(x, feature_embeddings, cls_token):
    batch = x.shape[0]
    num_feats, d = feature_embeddings.shape
    seq = num_feats + 1
    row = seq * d
    bpw = batch // _NW

    mesh = plsc.VectorSubcoreMesh(core_axis_name="c", subcore_axis_name="s")
    sc_fill = pl.kernel(
        functools.partial(_sc_body, row, bpw),
        out_type=jax.ShapeDtypeStruct((batch * row,), jnp.float32),
        mesh=mesh,
        scratch_types=[
            pltpu.VMEM((_R * row,), jnp.float32),
            pltpu.SemaphoreType.DMA,
        ],
    )
    out_flat = sc_fill(cls_token.reshape(d), feature_embeddings.reshape(num_feats * d))
    return out_flat.reshape(batch, seq, d)
